# baseline (speedup 1.0000x reference)
import functools
import jax
import jax.numpy as jnp
from jax import lax
from jax.experimental import pallas as pl
from jax.experimental.pallas import tpu as pltpu
from jax.experimental.pallas import tpu_sc as plsc

B, F, D = 16384, 100, 64
NC, NS, L = 2, 16, 16
NW = NC * NS
BPW = B // NW
CB = 8
NCH = BPW // CB
ROWS = CB * F
CHUNK_OUT = ROWS * D


def _sc_body(x_hbm, emb_hbm, w_hbm, bias_hbm, out_hbm,
             basebuf, wvbuf, biasbuf,
             xbuf0, xbuf1, obuf0, obuf1,
             sx0, sx1, so0, so1):
    wid = lax.axis_index("s") * NC + lax.axis_index("c")
    x0 = wid * (BPW * F)
    o0 = wid * (BPW * F * D)

    pltpu.sync_copy(emb_hbm, basebuf)
    pltpu.sync_copy(w_hbm, wvbuf)
    pltpu.sync_copy(bias_hbm, biasbuf)

    bias_regs = [biasbuf[pl.ds(dc * L, L)] for dc in range(4)]
    wv_regs = [wvbuf[pl.ds(dc * L, L)] for dc in range(4)]

    @plsc.parallel_loop(0, F)
    def _fold(f):
        for dc in range(4):
            o = f * D + dc * L
            basebuf[pl.ds(o, L)] = basebuf[pl.ds(o, L)] + bias_regs[dc]

    xbufs = (xbuf0, xbuf1)
    obufs = (obuf0, obuf1)
    sxs = (sx0, sx1)
    sos = (so0, so1)

    pltpu.async_copy(x_hbm.at[pl.ds(x0, ROWS)], xbuf0, sx0)
    pltpu.async_copy(x_hbm.at[pl.ds(x0 + ROWS, ROWS)], xbuf1, sx1)

    @pl.loop(0, NCH, step=2)
    def _chunks(c):
        for k in range(2):
            cc = c + k
            xb, ob, sx, so = xbufs[k], obufs[k], sxs[k], sos[k]
            pltpu.make_async_copy(
                x_hbm.at[pl.ds(x0 + cc * ROWS, ROWS)], xb, sx).wait()

            @pl.when(cc >= 2)
            def _():
                pltpu.make_async_copy(
                    ob, out_hbm.at[pl.ds(o0, CHUNK_OUT)], so).wait()

            @pl.loop(0, CB)
            def _batch(bi):
                row0 = bi * F

                @plsc.parallel_loop(0, F, unroll=2)
                def _row(f):
                    r = row0 + f
                    xs = plsc.load_gather(xb, [jnp.full((L,), r, jnp.int32)])
                    fb = f * D
                    ob_base = r * D
                    for dc in range(4):
                        ob[pl.ds(ob_base + dc * L, L)] = (
                            xs * wv_regs[dc] + basebuf[pl.ds(fb + dc * L, L)])

            pltpu.async_copy(
                ob, out_hbm.at[pl.ds(o0 + cc * CHUNK_OUT, CHUNK_OUT)], so)

            @pl.when(cc + 2 < NCH)
            def _():
                pltpu.async_copy(
                    x_hbm.at[pl.ds(x0 + (cc + 2) * ROWS, ROWS)], xb, sx)

    pltpu.make_async_copy(obuf0, out_hbm.at[pl.ds(o0, CHUNK_OUT)], so0).wait()
    pltpu.make_async_copy(obuf1, out_hbm.at[pl.ds(o0, CHUNK_OUT)], so1).wait()


@jax.jit
def kernel(x, emb_table, W, b):
    mesh = plsc.VectorSubcoreMesh(
        core_axis_name="c", subcore_axis_name="s",
        num_cores=NC, num_subcores=NS)
    out = pl.kernel(
        _sc_body,
        out_type=jax.ShapeDtypeStruct((B * F * D,), jnp.float32),
        mesh=mesh,
        scratch_types=[
            pltpu.VMEM((F * D,), jnp.float32),
            pltpu.VMEM((D,), jnp.float32),
            pltpu.VMEM((D,), jnp.float32),
            pltpu.VMEM((ROWS,), jnp.float32),
            pltpu.VMEM((ROWS,), jnp.float32),
            pltpu.VMEM((CHUNK_OUT,), jnp.float32),
            pltpu.VMEM((CHUNK_OUT,), jnp.float32),
            pltpu.SemaphoreType.DMA,
            pltpu.SemaphoreType.DMA,
            pltpu.SemaphoreType.DMA,
            pltpu.SemaphoreType.DMA,
        ],
        compiler_params=pltpu.CompilerParams(needs_layout_passes=False),
    )(x.reshape(-1), emb_table.reshape(-1), W.reshape(-1), b)
    return out.reshape(B, F, D)


# speedup vs baseline: 1.0017x; 1.0017x over previous
import functools
import jax
import jax.numpy as jnp
from jax import lax
from jax.experimental import pallas as pl
from jax.experimental import layout as jex_layout
from jax.experimental.pallas import tpu as pltpu
from jax.experimental.pallas import tpu_sc as plsc

B, F, D = 16384, 100, 64
NC, NS, L = 2, 16, 16
NW = NC * NS
BPW = B // NW
CB = 8
NCH = BPW // CB
ROWS = CB * F
CHUNK_OUT = ROWS * D


def _sc_body(x_hbm, emb_hbm, w_hbm, bias_hbm, out_hbm,
             basebuf, wvbuf, biasbuf,
             xbuf0, xbuf1, obuf0, obuf1,
             sx0, sx1, so0, so1):
    wid = lax.axis_index("s") * NC + lax.axis_index("c")
    x0 = wid * (BPW * F)
    o0 = wid * (BPW * F * D)

    pltpu.sync_copy(emb_hbm, basebuf)
    pltpu.sync_copy(w_hbm, wvbuf)
    pltpu.sync_copy(bias_hbm, biasbuf)

    bias_regs = [biasbuf[pl.ds(dc * L, L)] for dc in range(4)]
    wv_regs = [wvbuf[pl.ds(dc * L, L)] for dc in range(4)]

    @plsc.parallel_loop(0, F)
    def _fold(f):
        for dc in range(4):
            o = f * D + dc * L
            basebuf[pl.ds(o, L)] = basebuf[pl.ds(o, L)] + bias_regs[dc]

    xbufs = (xbuf0, xbuf1)
    obufs = (obuf0, obuf1)
    sxs = (sx0, sx1)
    sos = (so0, so1)

    pltpu.async_copy(x_hbm.at[pl.ds(x0, ROWS)], xbuf0, sx0)
    pltpu.async_copy(x_hbm.at[pl.ds(x0 + ROWS, ROWS)], xbuf1, sx1)

    @pl.loop(0, NCH, step=2)
    def _chunks(c):
        for k in range(2):
            cc = c + k
            xb, ob, sx, so = xbufs[k], obufs[k], sxs[k], sos[k]
            pltpu.make_async_copy(
                x_hbm.at[pl.ds(x0 + cc * ROWS, ROWS)], xb, sx).wait()

            @pl.when(cc >= 2)
            def _():
                pltpu.make_async_copy(
                    ob, out_hbm.at[pl.ds(o0, CHUNK_OUT)], so).wait()

            @pl.loop(0, CB)
            def _batch(bi):
                row0 = bi * F

                @plsc.parallel_loop(0, F, unroll=2)
                def _row(f):
                    r = row0 + f
                    xs = plsc.load_gather(xb, [jnp.full((L,), r, jnp.int32)])
                    fb = f * D
                    ob_base = r * D
                    for dc in range(4):
                        ob[pl.ds(ob_base + dc * L, L)] = (
                            xs * wv_regs[dc] + basebuf[pl.ds(fb + dc * L, L)])

            pltpu.async_copy(
                ob, out_hbm.at[pl.ds(o0 + cc * CHUNK_OUT, CHUNK_OUT)], so)

            @pl.when(cc + 2 < NCH)
            def _():
                pltpu.async_copy(
                    x_hbm.at[pl.ds(x0 + (cc + 2) * ROWS, ROWS)], xb, sx)

    pltpu.make_async_copy(obuf0, out_hbm.at[pl.ds(o0, CHUNK_OUT)], so0).wait()
    pltpu.make_async_copy(obuf1, out_hbm.at[pl.ds(o0, CHUNK_OUT)], so1).wait()


_JITTED = None


def kernel(x, emb_table, W, b):
    global _JITTED
    if _JITTED is None:
        fmt = jex_layout.Format(
            jex_layout.Layout(major_to_minor=(0, 1, 2), tiling=()),
            jax.sharding.SingleDeviceSharding(jax.devices()[0]))
        _JITTED = jax.jit(_kernel_impl, out_shardings=fmt)
    return _JITTED(x, emb_table, W, b)


def _kernel_impl(x, emb_table, W, b):
    mesh = plsc.VectorSubcoreMesh(
        core_axis_name="c", subcore_axis_name="s",
        num_cores=NC, num_subcores=NS)
    out = pl.kernel(
        _sc_body,
        out_type=jax.ShapeDtypeStruct((B * F * D,), jnp.float32),
        mesh=mesh,
        scratch_types=[
            pltpu.VMEM((F * D,), jnp.float32),
            pltpu.VMEM((D,), jnp.float32),
            pltpu.VMEM((D,), jnp.float32),
            pltpu.VMEM((ROWS,), jnp.float32),
            pltpu.VMEM((ROWS,), jnp.float32),
            pltpu.VMEM((CHUNK_OUT,), jnp.float32),
            pltpu.VMEM((CHUNK_OUT,), jnp.float32),
            pltpu.SemaphoreType.DMA,
            pltpu.SemaphoreType.DMA,
            pltpu.SemaphoreType.DMA,
            pltpu.SemaphoreType.DMA,
        ],
        compiler_params=pltpu.CompilerParams(needs_layout_passes=False),
    )(x.reshape(-1), emb_table.reshape(-1), W.reshape(-1), b)
    return out.reshape(B, F, D)
